# baseline (speedup 1.0000x reference)
import functools

import jax
import jax.numpy as jnp
from jax import lax
from jax.experimental import pallas as pl
from jax.experimental.pallas import tpu as pltpu
from jax.experimental.pallas import tpu_sc as plsc

EMBED = 64
FACTOR = 8.0

NUM_CORES = 2
NUM_SUBCORES = 16
NUM_WORKERS = NUM_CORES * NUM_SUBCORES

IDX_ROW = 128
ROWS_PER_GROUP = 4
CHUNK = IDX_ROW * ROWS_PER_GROUP


@functools.partial(jax.jit, static_argnums=(2, 3))
def _gather_scale(idx2d, table, n_idx_rows, rows_per_worker):
    groups_per_worker = rows_per_worker // ROWS_PER_GROUP
    total = n_idx_rows * IDX_ROW
    mesh = plsc.VectorSubcoreMesh(core_axis_name="c", subcore_axis_name="s")

    @functools.partial(
        pl.kernel,
        out_type=jax.ShapeDtypeStruct((total, EMBED), jnp.float32),
        mesh=mesh,
        scratch_types=[
            pltpu.VMEM((ROWS_PER_GROUP, IDX_ROW), jnp.int32),
            pltpu.VMEM((CHUNK, EMBED), jnp.float32),
            pltpu.SemaphoreType.DMA,
        ],
        compiler_params=pltpu.CompilerParams(use_tc_tiling_on_sc=False),
    )
    def body(table_hbm, idx_hbm, out_hbm, idx_v, rows_v, sem):
        wid = lax.axis_index("s") * NUM_CORES + lax.axis_index("c")
        row0 = wid * rows_per_worker
        out0 = row0 * IDX_ROW

        def group(g, carry):
            pltpu.sync_copy(
                idx_hbm.at[pl.ds(row0 + g * ROWS_PER_GROUP, ROWS_PER_GROUP)],
                idx_v,
            )
            copies = [
                pltpu.async_copy(
                    table_hbm.at[idx_v.at[j]],
                    rows_v.at[pl.ds(j * IDX_ROW, IDX_ROW)],
                    sem,
                )
                for j in range(ROWS_PER_GROUP)
            ]
            for cp in copies:
                cp.wait()

            def scale_row(r, c2):
                for c in range(EMBED // 16):
                    sl = pl.ds(c * 16, 16)
                    rows_v[r, sl] = rows_v[r, sl] * FACTOR
                return c2

            lax.fori_loop(0, CHUNK, scale_row, 0, unroll=2)

            pltpu.sync_copy(
                rows_v,
                out_hbm.at[pl.ds(out0 + g * CHUNK, CHUNK)],
            )
            return carry

        lax.fori_loop(0, groups_per_worker, group, 0)

    return body(table, idx2d)


def kernel(src, table):
    b0, b1 = src.shape
    total = b0 * b1
    assert total % (NUM_WORKERS * CHUNK) == 0
    idx2d = src.reshape(total // IDX_ROW, IDX_ROW)
    rows_per_worker = (total // IDX_ROW) // NUM_WORKERS
    out = _gather_scale(idx2d, table, total // IDX_ROW, rows_per_worker)
    return out.reshape(b0, b1, EMBED)


# speedup vs baseline: 1.0723x; 1.0723x over previous
import functools

import jax
import jax.numpy as jnp
from jax import lax
from jax.experimental import pallas as pl
from jax.experimental.pallas import tpu as pltpu
from jax.experimental.pallas import tpu_sc as plsc

EMBED = 64
FACTOR = 8.0

NUM_CORES = 2
NUM_SUBCORES = 16
NUM_WORKERS = NUM_CORES * NUM_SUBCORES

IDX_ROW = 128
ROWS_PER_GROUP = 4
CHUNK = IDX_ROW * ROWS_PER_GROUP


@functools.partial(jax.jit, static_argnums=(2, 3))
def _gather_scale(idx2d, table, n_idx_rows, rows_per_worker):
    groups_per_worker = rows_per_worker // ROWS_PER_GROUP
    assert groups_per_worker % 2 == 0
    total = n_idx_rows * IDX_ROW
    mesh = plsc.VectorSubcoreMesh(core_axis_name="c", subcore_axis_name="s")

    @functools.partial(
        pl.kernel,
        out_type=jax.ShapeDtypeStruct((total, EMBED), jnp.float32),
        mesh=mesh,
        scratch_types=[
            pltpu.VMEM((ROWS_PER_GROUP, IDX_ROW), jnp.int32),
            pltpu.VMEM((ROWS_PER_GROUP, IDX_ROW), jnp.int32),
            pltpu.VMEM((CHUNK, EMBED), jnp.float32),
            pltpu.VMEM((CHUNK, EMBED), jnp.float32),
            pltpu.SemaphoreType.DMA,
            pltpu.SemaphoreType.DMA,
            pltpu.SemaphoreType.DMA,
            pltpu.SemaphoreType.DMA,
        ],
        compiler_params=pltpu.CompilerParams(use_tc_tiling_on_sc=False),
    )
    def body(table_hbm, idx_hbm, out_hbm, idx0, idx1, rows0, rows1,
             gsem0, gsem1, wsem0, wsem1):
        wid = lax.axis_index("s") * NUM_CORES + lax.axis_index("c")
        row0 = wid * rows_per_worker
        out0 = row0 * IDX_ROW

        def stage_and_fire(g, idx_v, rows_v, gsem):
            pltpu.sync_copy(
                idx_hbm.at[pl.ds(row0 + g * ROWS_PER_GROUP, ROWS_PER_GROUP)],
                idx_v,
            )
            for j in range(ROWS_PER_GROUP):
                pltpu.async_copy(
                    table_hbm.at[idx_v.at[j]],
                    rows_v.at[pl.ds(j * IDX_ROW, IDX_ROW)],
                    gsem,
                )

        def drain_gather(rows_v, gsem):
            pltpu.make_async_copy(
                out_hbm.at[pl.ds(0, CHUNK)], rows_v, gsem
            ).wait()

        def drain_write(g, rows_v, wsem):
            pltpu.make_async_copy(
                rows_v, out_hbm.at[pl.ds(out0 + g * CHUNK, CHUNK)], wsem
            ).wait()

        def scale(rows_v):
            def scale_row(r, c2):
                for c in range(EMBED // 16):
                    sl = pl.ds(c * 16, 16)
                    rows_v[r, sl] = rows_v[r, sl] * FACTOR
                return c2

            lax.fori_loop(0, CHUNK, scale_row, 0, unroll=2)

        def fire_write(g, rows_v, wsem):
            pltpu.async_copy(
                rows_v, out_hbm.at[pl.ds(out0 + g * CHUNK, CHUNK)], wsem
            )

        stage_and_fire(0, idx0, rows0, gsem0)

        def step(i, carry):
            g0 = 2 * i
            g1 = g0 + 1

            drain_gather(rows0, gsem0)

            @pl.when(i > 0)
            def _():
                drain_write(g0 - 1, rows1, wsem1)

            stage_and_fire(g1, idx1, rows1, gsem1)
            scale(rows0)
            fire_write(g0, rows0, wsem0)

            drain_gather(rows1, gsem1)

            @pl.when(g1 + 1 < groups_per_worker)
            def _():
                drain_write(g0, rows0, wsem0)
                stage_and_fire(g1 + 1, idx0, rows0, gsem0)

            scale(rows1)
            fire_write(g1, rows1, wsem1)
            return carry

        lax.fori_loop(0, groups_per_worker // 2, step, 0)

        drain_write(groups_per_worker - 2, rows0, wsem0)
        drain_write(groups_per_worker - 1, rows1, wsem1)

    return body(table, idx2d)


def kernel(src, table):
    b0, b1 = src.shape
    total = b0 * b1
    assert total % (NUM_WORKERS * CHUNK) == 0
    idx2d = src.reshape(total // IDX_ROW, IDX_ROW)
    rows_per_worker = (total // IDX_ROW) // NUM_WORKERS
    out = _gather_scale(idx2d, table, total // IDX_ROW, rows_per_worker)
    return out.reshape(b0, b1, EMBED)


# speedup vs baseline: 1.0868x; 1.0135x over previous
import functools

import jax
import jax.numpy as jnp
from jax import lax
from jax.experimental import pallas as pl
from jax.experimental.pallas import tpu as pltpu
from jax.experimental.pallas import tpu_sc as plsc

EMBED = 64
FACTOR = 8.0

NUM_CORES = 2
NUM_SUBCORES = 16
NUM_WORKERS = NUM_CORES * NUM_SUBCORES

IDX_SPLIT = 128
ROWS_PER_GROUP = 4


@functools.partial(jax.jit, static_argnums=(2,))
def _gather_scale(src, table, rows_per_worker):
    n_rows, row_len = src.shape
    groups_per_worker = rows_per_worker // ROWS_PER_GROUP
    assert groups_per_worker % 2 == 0
    tail = row_len - IDX_SPLIT
    assert 0 < tail <= IDX_SPLIT and IDX_SPLIT % 8 == 0
    mesh = plsc.VectorSubcoreMesh(core_axis_name="c", subcore_axis_name="s")

    @functools.partial(
        pl.kernel,
        out_type=jax.ShapeDtypeStruct((n_rows, row_len, EMBED), jnp.float32),
        mesh=mesh,
        scratch_types=[
            pltpu.VMEM((ROWS_PER_GROUP, row_len), jnp.int32),
            pltpu.VMEM((ROWS_PER_GROUP, row_len), jnp.int32),
            pltpu.VMEM((ROWS_PER_GROUP, row_len, EMBED), jnp.float32),
            pltpu.VMEM((ROWS_PER_GROUP, row_len, EMBED), jnp.float32),
            pltpu.SemaphoreType.DMA,
            pltpu.SemaphoreType.DMA,
            pltpu.SemaphoreType.DMA,
            pltpu.SemaphoreType.DMA,
        ],
        compiler_params=pltpu.CompilerParams(use_tc_tiling_on_sc=False),
    )
    def body(table_hbm, src_hbm, out_hbm, idx0, idx1, rows0, rows1,
             gsem0, gsem1, wsem0, wsem1):
        wid = lax.axis_index("s") * NUM_CORES + lax.axis_index("c")
        row0 = wid * rows_per_worker

        def stage_and_fire(g, idx_v, rows_v, gsem):
            pltpu.sync_copy(
                src_hbm.at[pl.ds(row0 + g * ROWS_PER_GROUP, ROWS_PER_GROUP)],
                idx_v,
            )
            for j in range(ROWS_PER_GROUP):
                pltpu.async_copy(
                    table_hbm.at[idx_v.at[j, pl.ds(0, IDX_SPLIT)]],
                    rows_v.at[j, pl.ds(0, IDX_SPLIT)],
                    gsem,
                )
                pltpu.async_copy(
                    table_hbm.at[idx_v.at[j, pl.ds(IDX_SPLIT, tail)]],
                    rows_v.at[j, pl.ds(IDX_SPLIT, tail)],
                    gsem,
                )

        def drain_gather(rows_v, gsem):
            pltpu.make_async_copy(
                out_hbm.at[pl.ds(0, ROWS_PER_GROUP)], rows_v, gsem
            ).wait()

        def drain_write(g, rows_v, wsem):
            pltpu.make_async_copy(
                rows_v,
                out_hbm.at[pl.ds(row0 + g * ROWS_PER_GROUP, ROWS_PER_GROUP)],
                wsem,
            ).wait()

        def scale(rows_v):
            def scale_col(c, c2):
                for j in range(ROWS_PER_GROUP):
                    for k in range(EMBED // 16):
                        sl = pl.ds(k * 16, 16)
                        rows_v[j, c, sl] = rows_v[j, c, sl] * FACTOR
                return c2

            lax.fori_loop(0, row_len, scale_col, 0, unroll=2)

        def fire_write(g, rows_v, wsem):
            pltpu.async_copy(
                rows_v,
                out_hbm.at[pl.ds(row0 + g * ROWS_PER_GROUP, ROWS_PER_GROUP)],
                wsem,
            )

        stage_and_fire(0, idx0, rows0, gsem0)

        def step(i, carry):
            g0 = 2 * i
            g1 = g0 + 1

            drain_gather(rows0, gsem0)

            @pl.when(i > 0)
            def _():
                drain_write(g0 - 1, rows1, wsem1)

            stage_and_fire(g1, idx1, rows1, gsem1)
            scale(rows0)
            fire_write(g0, rows0, wsem0)

            drain_gather(rows1, gsem1)

            @pl.when(g1 + 1 < groups_per_worker)
            def _():
                drain_write(g0, rows0, wsem0)
                stage_and_fire(g1 + 1, idx0, rows0, gsem0)

            scale(rows1)
            fire_write(g1, rows1, wsem1)
            return carry

        lax.fori_loop(0, groups_per_worker // 2, step, 0)

        drain_write(groups_per_worker - 2, rows0, wsem0)
        drain_write(groups_per_worker - 1, rows1, wsem1)

    return body(table, src)


def kernel(src, table):
    n_rows, _ = src.shape
    assert n_rows % NUM_WORKERS == 0
    return _gather_scale(src, table, n_rows // NUM_WORKERS)
